# Initial kernel scaffold; baseline (speedup 1.0000x reference)
import jax
import jax.numpy as jnp
from jax.experimental import pallas as pl


def kernel(speed, quats, times_dif, dir, mes, weight, bias, bias_shift, time_shift, types):
    raise NotImplementedError("write your pallas kernel here")


# speedup vs baseline: 2.3549x; 2.3549x over previous
import numpy as np
import jax
import jax.numpy as jnp
from jax.experimental import pallas as pl

_R = 256
_N = 256


def _block_kernel(ts_ref, bsh_ref, types_ref,
                  s0_ref, s1_ref, s2_ref, d0_ref, d1_ref, d2_ref,
                  dir_ref, mes_ref, w_ref, btc_ref, btn_ref,
                  out_ref):
    ts0 = ts_ref[0, 0]
    ts1 = ts_ref[0, 1]
    ts2 = ts_ref[0, 2]
    inv_sum = 1.0 / (ts0 + ts1 + ts2)
    sm = (s0_ref[...] * (1000.0 / d0_ref[...]) * ts0
          + s1_ref[...] * (1000.0 / d1_ref[...]) * ts1
          + s2_ref[...] * (1000.0 / d2_ref[...]) * ts2) * inv_sum
    sm = sm + bsh_ref[...] * 0.01

    lane = jax.lax.broadcasted_iota(jnp.int32, (3, 3 * _N), 1)
    comp = jax.lax.broadcasted_iota(jnp.int32, (3, 3 * _N), 0)
    expand = (lane % 3 == comp).astype(jnp.float32)
    sm_t = jnp.dot(sm, expand, preferred_element_type=jnp.float32)

    row3 = jax.lax.broadcasted_iota(jnp.int32, (3 * _N, _N), 0)
    col3 = jax.lax.broadcasted_iota(jnp.int32, (3 * _N, _N), 1)
    fold = (row3 // 3 == col3).astype(jnp.float32)
    dot = jnp.dot(dir_ref[...] * sm_t, fold,
                  preferred_element_type=jnp.float32)

    krow = jax.lax.broadcasted_iota(jnp.int32, (6, _N), 0)
    onehot = (types_ref[...] == krow).astype(jnp.float32)
    b = jnp.dot(btc_ref[...], onehot, preferred_element_type=jnp.float32)

    w = w_ref[...]
    mes_est = dot - mes_ref[...] + b

    x = mes_est + (w == 0.0).astype(jnp.float32) * 10000000.0
    u = jax.lax.bitcast_convert_type(x, jnp.int32)
    skey = u ^ ((u >> 31) & jnp.int32(0x7FFFFFFF))
    ind = jnp.sum((w > 0.0).astype(jnp.int32), axis=-1, keepdims=True)
    k = ind // 2
    p = jnp.full((x.shape[0], 1), np.int32(-2147483648), jnp.int32)
    for bit in range(31, -1, -1):
        inc = np.int32(-2147483648) if bit == 31 else np.int32(1 << bit)
        cand = p + inc
        cnt = jnp.sum((skey < cand).astype(jnp.int32), axis=-1, keepdims=True)
        p = jnp.where(cnt <= k, cand, p)
    med_u = p ^ ((p >> 31) & jnp.int32(0x7FFFFFFF))
    med = jax.lax.bitcast_convert_type(med_u, jnp.float32)
    med = med * (ind > 0).astype(jnp.float32)

    row = jnp.sum(jnp.abs(mes_est - med) * w, axis=-1, keepdims=True)
    row = row * np.float32(1.0 / _N)
    bias_loss = jnp.sum(jnp.abs(btn_ref[...] - btc_ref[...]), axis=-1,
                        keepdims=True)
    out_ref[...] = row + bias_loss


def kernel(speed, quats, times_dif, dir, mes, weight, bias, bias_shift,
           time_shift, types):
    del quats
    tp = dir.shape[0]
    sp_ext = jnp.concatenate([speed, speed[-1:]], axis=0)
    dif_ext = jnp.concatenate([times_dif, times_dif[-1:]], axis=0)
    s0, s1, s2 = sp_ext[2:], sp_ext[1:-1], sp_ext[:-2]
    d0, d1, d2 = dif_ext[2:], dif_ext[1:-1], dif_ext[:-2]
    dir2 = dir.reshape(tp, 3 * _N)
    bias_t = jnp.transpose(bias)
    bias_tn = jnp.concatenate([bias_t[1:], bias_t[-1:]], axis=0)
    ts = time_shift.reshape(1, 3)
    types2 = types.reshape(1, _N)

    grid = (tp + _R - 1) // _R
    row_spec3 = pl.BlockSpec((_R, 3), lambda i: (i, 0))
    row_spec1 = pl.BlockSpec((_R, 1), lambda i: (i, 0))
    out = pl.pallas_call(
        _block_kernel,
        grid=(grid,),
        in_specs=[
            pl.BlockSpec((1, 3), lambda i: (0, 0)),
            pl.BlockSpec((1, 3), lambda i: (0, 0)),
            pl.BlockSpec((1, _N), lambda i: (0, 0)),
            row_spec3, row_spec3, row_spec3,
            row_spec1, row_spec1, row_spec1,
            pl.BlockSpec((_R, 3 * _N), lambda i: (i, 0)),
            pl.BlockSpec((_R, _N), lambda i: (i, 0)),
            pl.BlockSpec((_R, _N), lambda i: (i, 0)),
            pl.BlockSpec((_R, 6), lambda i: (i, 0)),
            pl.BlockSpec((_R, 6), lambda i: (i, 0)),
        ],
        out_specs=pl.BlockSpec((_R, 1), lambda i: (i, 0)),
        out_shape=jax.ShapeDtypeStruct((tp, 1), jnp.float32),
    )(ts, bias_shift, types2, s0, s1, s2, d0, d1, d2, dir2, mes, weight,
      bias_t, bias_tn)
    loss = out.reshape(tp)
    return jnp.concatenate([jnp.zeros((1,), jnp.float32), loss], axis=0)


# speedup vs baseline: 2.4357x; 1.0343x over previous
import numpy as np
import jax
import jax.numpy as jnp
from jax.experimental import pallas as pl

_R = 256
_N = 256


def _block_kernel(ts_ref, bsh_ref, types_ref,
                  s0_ref, s1_ref, s2_ref, d0_ref, d1_ref, d2_ref,
                  dir_ref, mes_ref, w_ref, btc_ref, btn_ref,
                  out_ref):
    ts0 = ts_ref[0, 0]
    ts1 = ts_ref[0, 1]
    ts2 = ts_ref[0, 2]
    inv_sum = 1.0 / (ts0 + ts1 + ts2)
    sm = (s0_ref[...] * (1000.0 / d0_ref[...]) * ts0
          + s1_ref[...] * (1000.0 / d1_ref[...]) * ts1
          + s2_ref[...] * (1000.0 / d2_ref[...]) * ts2) * inv_sum
    sm = sm + bsh_ref[...] * 0.01

    lane = jax.lax.broadcasted_iota(jnp.int32, (3, 3 * _N), 1)
    comp = jax.lax.broadcasted_iota(jnp.int32, (3, 3 * _N), 0)
    expand = (lane % 3 == comp).astype(jnp.float32)
    sm_t = jnp.dot(sm, expand, preferred_element_type=jnp.float32)

    row3 = jax.lax.broadcasted_iota(jnp.int32, (3 * _N, _N), 0)
    col3 = jax.lax.broadcasted_iota(jnp.int32, (3 * _N, _N), 1)
    fold = (row3 // 3 == col3).astype(jnp.float32)
    dot = jnp.dot(dir_ref[...] * sm_t, fold,
                  preferred_element_type=jnp.float32)

    krow = jax.lax.broadcasted_iota(jnp.int32, (6, _N), 0)
    onehot = (types_ref[...] == krow).astype(jnp.float32)
    b = jnp.dot(btc_ref[...], onehot, preferred_element_type=jnp.float32)

    w = w_ref[...]
    mes_est = dot - mes_ref[...] + b

    x = mes_est + (w == 0.0).astype(jnp.float32) * 10000000.0
    u = jax.lax.bitcast_convert_type(x, jnp.int32)
    skey = u ^ ((u >> 31) & jnp.int32(0x7FFFFFFF))
    ind = jnp.sum((w > 0.0).astype(jnp.int32), axis=-1, keepdims=True)
    kf = (ind // 2).astype(jnp.float32)
    p = jnp.full((x.shape[0], 1), np.int32(-2147483648), jnp.int32)
    for bit in range(31, -1, -1):
        inc = np.int32(-2147483648) if bit == 31 else np.int32(1 << bit)
        cand = p + inc
        cnt = jnp.sum(jnp.where(skey < cand, 1.0, 0.0), axis=-1,
                      keepdims=True)
        p = jnp.where(cnt <= kf, cand, p)
    med_u = p ^ ((p >> 31) & jnp.int32(0x7FFFFFFF))
    med = jax.lax.bitcast_convert_type(med_u, jnp.float32)
    med = med * (ind > 0).astype(jnp.float32)

    row = jnp.sum(jnp.abs(mes_est - med) * w, axis=-1, keepdims=True)
    row = row * np.float32(1.0 / _N)
    bias_loss = jnp.sum(jnp.abs(btn_ref[...] - btc_ref[...]), axis=-1,
                        keepdims=True)
    out_ref[...] = row + bias_loss


def kernel(speed, quats, times_dif, dir, mes, weight, bias, bias_shift,
           time_shift, types):
    del quats
    tp = dir.shape[0]
    sp_ext = jnp.concatenate([speed, speed[-1:]], axis=0)
    dif_ext = jnp.concatenate([times_dif, times_dif[-1:]], axis=0)
    s0, s1, s2 = sp_ext[2:], sp_ext[1:-1], sp_ext[:-2]
    d0, d1, d2 = dif_ext[2:], dif_ext[1:-1], dif_ext[:-2]
    dir2 = dir.reshape(tp, 3 * _N)
    bias_t = jnp.transpose(bias)
    bias_tn = jnp.concatenate([bias_t[1:], bias_t[-1:]], axis=0)
    ts = time_shift.reshape(1, 3)
    types2 = types.reshape(1, _N)

    grid = (tp + _R - 1) // _R
    row_spec3 = pl.BlockSpec((_R, 3), lambda i: (i, 0))
    row_spec1 = pl.BlockSpec((_R, 1), lambda i: (i, 0))
    out = pl.pallas_call(
        _block_kernel,
        grid=(grid,),
        in_specs=[
            pl.BlockSpec((1, 3), lambda i: (0, 0)),
            pl.BlockSpec((1, 3), lambda i: (0, 0)),
            pl.BlockSpec((1, _N), lambda i: (0, 0)),
            row_spec3, row_spec3, row_spec3,
            row_spec1, row_spec1, row_spec1,
            pl.BlockSpec((_R, 3 * _N), lambda i: (i, 0)),
            pl.BlockSpec((_R, _N), lambda i: (i, 0)),
            pl.BlockSpec((_R, _N), lambda i: (i, 0)),
            pl.BlockSpec((_R, 6), lambda i: (i, 0)),
            pl.BlockSpec((_R, 6), lambda i: (i, 0)),
        ],
        out_specs=pl.BlockSpec((_R, 1), lambda i: (i, 0)),
        out_shape=jax.ShapeDtypeStruct((tp, 1), jnp.float32),
    )(ts, bias_shift, types2, s0, s1, s2, d0, d1, d2, dir2, mes, weight,
      bias_t, bias_tn)
    loss = out.reshape(tp)
    return jnp.concatenate([jnp.zeros((1,), jnp.float32), loss], axis=0)


# speedup vs baseline: 3.9253x; 1.6116x over previous
import numpy as np
import jax
import jax.numpy as jnp
from jax.experimental import pallas as pl

_R = 256
_N = 256


def _block_kernel(ts_ref, bsh_ref, types_ref,
                  s0_ref, s1_ref, s2_ref, d0_ref, d1_ref, d2_ref,
                  dir_ref, mes_ref, w_ref, btc_ref, btn_ref,
                  out_ref):
    ts0 = ts_ref[0, 0]
    ts1 = ts_ref[0, 1]
    ts2 = ts_ref[0, 2]
    inv_sum = 1.0 / (ts0 + ts1 + ts2)
    sm = (s0_ref[...] * (1000.0 / d0_ref[...]) * ts0
          + s1_ref[...] * (1000.0 / d1_ref[...]) * ts1
          + s2_ref[...] * (1000.0 / d2_ref[...]) * ts2) * inv_sum
    sm = sm + bsh_ref[...] * 0.01

    dot = (dir_ref[0] * sm[:, 0:1] + dir_ref[1] * sm[:, 1:2]
           + dir_ref[2] * sm[:, 2:3])

    krow = jax.lax.broadcasted_iota(jnp.int32, (6, _N), 0)
    onehot = (types_ref[...] == krow).astype(jnp.float32)
    b = jnp.dot(btc_ref[...], onehot, preferred_element_type=jnp.float32)

    w = w_ref[...]
    mes_est = dot - mes_ref[...] + b

    x = mes_est + (w == 0.0).astype(jnp.float32) * 10000000.0
    u = jax.lax.bitcast_convert_type(x, jnp.int32)
    skey = u ^ ((u >> 31) & jnp.int32(0x7FFFFFFF))
    ind = jnp.sum((w > 0.0).astype(jnp.int32), axis=-1, keepdims=True)
    kf = (ind // 2).astype(jnp.float32)
    p = jnp.full((x.shape[0], 1), np.int32(-2147483648), jnp.int32)
    for bit in range(31, -1, -1):
        inc = np.int32(-2147483648) if bit == 31 else np.int32(1 << bit)
        cand = p + inc
        cnt = jnp.sum(jnp.where(skey < cand, 1.0, 0.0), axis=-1,
                      keepdims=True)
        p = jnp.where(cnt <= kf, cand, p)
    med_u = p ^ ((p >> 31) & jnp.int32(0x7FFFFFFF))
    med = jax.lax.bitcast_convert_type(med_u, jnp.float32)
    med = med * (ind > 0).astype(jnp.float32)

    row = jnp.sum(jnp.abs(mes_est - med) * w, axis=-1, keepdims=True)
    row = row * np.float32(1.0 / _N)
    bias_loss = jnp.sum(jnp.abs(btn_ref[...] - btc_ref[...]), axis=-1,
                        keepdims=True)
    out_ref[...] = row + bias_loss


def kernel(speed, quats, times_dif, dir, mes, weight, bias, bias_shift,
           time_shift, types):
    del quats
    tp = dir.shape[0]
    sp_ext = jnp.concatenate([speed, speed[-1:]], axis=0)
    dif_ext = jnp.concatenate([times_dif, times_dif[-1:]], axis=0)
    s0, s1, s2 = sp_ext[2:], sp_ext[1:-1], sp_ext[:-2]
    d0, d1, d2 = dif_ext[2:], dif_ext[1:-1], dif_ext[:-2]
    dir_p = jnp.transpose(dir, (2, 0, 1))
    bias_t = jnp.transpose(bias)
    bias_tn = jnp.concatenate([bias_t[1:], bias_t[-1:]], axis=0)
    ts = time_shift.reshape(1, 3)
    types2 = types.reshape(1, _N)

    grid = (tp + _R - 1) // _R
    row_spec3 = pl.BlockSpec((_R, 3), lambda i: (i, 0))
    row_spec1 = pl.BlockSpec((_R, 1), lambda i: (i, 0))
    out = pl.pallas_call(
        _block_kernel,
        grid=(grid,),
        in_specs=[
            pl.BlockSpec((1, 3), lambda i: (0, 0)),
            pl.BlockSpec((1, 3), lambda i: (0, 0)),
            pl.BlockSpec((1, _N), lambda i: (0, 0)),
            row_spec3, row_spec3, row_spec3,
            row_spec1, row_spec1, row_spec1,
            pl.BlockSpec((3, _R, _N), lambda i: (0, i, 0)),
            pl.BlockSpec((_R, _N), lambda i: (i, 0)),
            pl.BlockSpec((_R, _N), lambda i: (i, 0)),
            pl.BlockSpec((_R, 6), lambda i: (i, 0)),
            pl.BlockSpec((_R, 6), lambda i: (i, 0)),
        ],
        out_specs=pl.BlockSpec((_R, 1), lambda i: (i, 0)),
        out_shape=jax.ShapeDtypeStruct((tp, 1), jnp.float32),
    )(ts, bias_shift, types2, s0, s1, s2, d0, d1, d2, dir_p, mes, weight,
      bias_t, bias_tn)
    loss = out.reshape(tp)
    return jnp.concatenate([jnp.zeros((1,), jnp.float32), loss], axis=0)


# speedup vs baseline: 4.9122x; 1.2514x over previous
import functools

import numpy as np
import jax
import jax.numpy as jnp
from jax.experimental import pallas as pl

_R = 256
_N = 256


def _block_kernel(ts_ref, bsh_ref, types_ref,
                  s0_ref, s1_ref, s2_ref, d0_ref, d1_ref, d2_ref,
                  dir_ref, mes_ref, w_ref, bc_ref, bn_ref,
                  out_ref, *, tp):
    ts0 = ts_ref[0, 0]
    ts1 = ts_ref[0, 1]
    ts2 = ts_ref[0, 2]
    inv_sum = 1.0 / (ts0 + ts1 + ts2)
    sm = (s0_ref[...] * (1000.0 / d0_ref[...]) * ts0
          + s1_ref[...] * (1000.0 / d1_ref[...]) * ts1
          + s2_ref[...] * (1000.0 / d2_ref[...]) * ts2) * inv_sum
    sm = sm + bsh_ref[...] * 0.01

    c = (dir_ref[0] * sm[:, 0:1] + dir_ref[1] * sm[:, 1:2]
         + dir_ref[2] * sm[:, 2:3]) - mes_ref[...]

    row0 = pl.program_id(0) * _R
    valid = (jax.lax.broadcasted_iota(jnp.int32, (_R, _N), 0) + row0
             < tp)
    c = jnp.where(valid, c, 0.0)
    w_in = jnp.where(valid, w_ref[...], 0.0)

    ident = (jax.lax.broadcasted_iota(jnp.int32, (_R, _R), 0)
             == jax.lax.broadcasted_iota(jnp.int32, (_R, _R), 1)
             ).astype(jnp.float32)
    tdims = (((0,), (0,)), ((), ()))
    c_t = jax.lax.dot_general(c, ident, tdims,
                              preferred_element_type=jnp.float32)
    w_t = jax.lax.dot_general(w_in, ident, tdims,
                              preferred_element_type=jnp.float32)

    kcol = jax.lax.broadcasted_iota(jnp.int32, (_N, 6), 1)
    onehot = (types_ref[...] == kcol).astype(jnp.float32)
    b_t = jnp.dot(onehot, bc_ref[...], preferred_element_type=jnp.float32)

    mes_est = c_t + b_t

    x = mes_est + (w_t == 0.0).astype(jnp.float32) * 10000000.0
    u = jax.lax.bitcast_convert_type(x, jnp.int32)
    skey = u ^ ((u >> 31) & jnp.int32(0x7FFFFFFF))
    ind = jnp.sum((w_t > 0.0).astype(jnp.int32), axis=0, keepdims=True)
    kf = (ind // 2).astype(jnp.float32)
    p = jnp.full((1, _R), np.int32(-2147483648), jnp.int32)
    for bit in range(31, -1, -1):
        inc = np.int32(-2147483648) if bit == 31 else np.int32(1 << bit)
        cand = p + inc
        cnt = jnp.sum(jnp.where(skey < cand, 1.0, 0.0), axis=0,
                      keepdims=True)
        p = jnp.where(cnt <= kf, cand, p)
    med_u = p ^ ((p >> 31) & jnp.int32(0x7FFFFFFF))
    med = jax.lax.bitcast_convert_type(med_u, jnp.float32)
    med = med * (ind > 0).astype(jnp.float32)

    row = jnp.sum(jnp.abs(mes_est - med) * w_t, axis=0, keepdims=True)
    row = row * np.float32(1.0 / _N)
    bias_loss = jnp.sum(jnp.abs(bn_ref[...] - bc_ref[...]), axis=0,
                        keepdims=True)
    out_ref[...] = row + bias_loss


def kernel(speed, quats, times_dif, dir, mes, weight, bias, bias_shift,
           time_shift, types):
    del quats
    tp = dir.shape[0]
    sp_ext = jnp.concatenate([speed, speed[-1:]], axis=0)
    dif_ext = jnp.concatenate([times_dif, times_dif[-1:]], axis=0)
    s0, s1, s2 = sp_ext[2:], sp_ext[1:-1], sp_ext[:-2]
    d0, d1, d2 = dif_ext[2:], dif_ext[1:-1], dif_ext[:-2]
    dir_p = jnp.transpose(dir, (2, 0, 1))
    bias_n = jnp.concatenate([bias[:, 1:], bias[:, -1:]], axis=1)
    ts = time_shift.reshape(1, 3)
    types2 = types.reshape(_N, 1)

    grid = (tp + _R - 1) // _R
    row_spec3 = pl.BlockSpec((_R, 3), lambda i: (i, 0))
    row_spec1 = pl.BlockSpec((_R, 1), lambda i: (i, 0))
    out = pl.pallas_call(
        functools.partial(_block_kernel, tp=tp),
        grid=(grid,),
        in_specs=[
            pl.BlockSpec((1, 3), lambda i: (0, 0)),
            pl.BlockSpec((1, 3), lambda i: (0, 0)),
            pl.BlockSpec((_N, 1), lambda i: (0, 0)),
            row_spec3, row_spec3, row_spec3,
            row_spec1, row_spec1, row_spec1,
            pl.BlockSpec((3, _R, _N), lambda i: (0, i, 0)),
            pl.BlockSpec((_R, _N), lambda i: (i, 0)),
            pl.BlockSpec((_R, _N), lambda i: (i, 0)),
            pl.BlockSpec((6, _R), lambda i: (0, i)),
            pl.BlockSpec((6, _R), lambda i: (0, i)),
        ],
        out_specs=pl.BlockSpec((1, _R), lambda i: (0, i)),
        out_shape=jax.ShapeDtypeStruct((1, tp), jnp.float32),
    )(ts, bias_shift, types2, s0, s1, s2, d0, d1, d2, dir_p, mes, weight,
      bias, bias_n)
    loss = out.reshape(tp)
    return jnp.concatenate([jnp.zeros((1,), jnp.float32), loss], axis=0)


# speedup vs baseline: 6.1232x; 1.2465x over previous
import functools

import numpy as np
import jax
import jax.numpy as jnp
from jax.experimental import pallas as pl

_R = 1024
_N = 256


def _block_kernel(ts_ref, bsh_ref, types_ref,
                  s0_ref, s1_ref, s2_ref, d0_ref, d1_ref, d2_ref,
                  dir_ref, mes_ref, w_ref, bc_ref, bn_ref,
                  out_ref, *, tp):
    ts0 = ts_ref[0, 0]
    ts1 = ts_ref[0, 1]
    ts2 = ts_ref[0, 2]
    inv_sum = 1.0 / (ts0 + ts1 + ts2)
    sm = (s0_ref[...] * (1000.0 / d0_ref[...]) * ts0
          + s1_ref[...] * (1000.0 / d1_ref[...]) * ts1
          + s2_ref[...] * (1000.0 / d2_ref[...]) * ts2) * inv_sum
    sm = sm + bsh_ref[...] * 0.01

    c = (dir_ref[0] * sm[:, 0:1] + dir_ref[1] * sm[:, 1:2]
         + dir_ref[2] * sm[:, 2:3]) - mes_ref[...]

    row0 = pl.program_id(0) * _R
    valid = (jax.lax.broadcasted_iota(jnp.int32, (_R, _N), 0) + row0
             < tp)
    c = jnp.where(valid, c, 0.0)
    w_in = jnp.where(valid, w_ref[...], 0.0)

    ident = (jax.lax.broadcasted_iota(jnp.int32, (_R, _R), 0)
             == jax.lax.broadcasted_iota(jnp.int32, (_R, _R), 1)
             ).astype(jnp.float32)
    tdims = (((0,), (0,)), ((), ()))
    c_t = jax.lax.dot_general(c, ident, tdims,
                              preferred_element_type=jnp.float32)
    w_t = jax.lax.dot_general(w_in, ident, tdims,
                              preferred_element_type=jnp.float32)

    kcol = jax.lax.broadcasted_iota(jnp.int32, (_N, 6), 1)
    onehot = (types_ref[...] == kcol).astype(jnp.float32)
    b_t = jnp.dot(onehot, bc_ref[...], preferred_element_type=jnp.float32)

    mes_est = c_t + b_t

    x = mes_est + (w_t == 0.0).astype(jnp.float32) * 10000000.0
    u = jax.lax.bitcast_convert_type(x, jnp.int32)
    skey = u ^ ((u >> 31) & jnp.int32(0x7FFFFFFF))
    ind = jnp.sum((w_t > 0.0).astype(jnp.int32), axis=0, keepdims=True)
    kf = (ind // 2).astype(jnp.float32)
    p = jnp.full((1, _R), np.int32(-2147483648), jnp.int32)
    for bit in range(31, -1, -1):
        inc = np.int32(-2147483648) if bit == 31 else np.int32(1 << bit)
        cand = p + inc
        cnt = jnp.sum(jnp.where(skey < cand, 1.0, 0.0), axis=0,
                      keepdims=True)
        p = jnp.where(cnt <= kf, cand, p)
    med_u = p ^ ((p >> 31) & jnp.int32(0x7FFFFFFF))
    med = jax.lax.bitcast_convert_type(med_u, jnp.float32)
    med = med * (ind > 0).astype(jnp.float32)

    row = jnp.sum(jnp.abs(mes_est - med) * w_t, axis=0, keepdims=True)
    row = row * np.float32(1.0 / _N)
    bias_loss = jnp.sum(jnp.abs(bn_ref[...] - bc_ref[...]), axis=0,
                        keepdims=True)
    out_ref[...] = row + bias_loss


def kernel(speed, quats, times_dif, dir, mes, weight, bias, bias_shift,
           time_shift, types):
    del quats
    tp = dir.shape[0]
    sp_ext = jnp.concatenate([speed, speed[-1:]], axis=0)
    dif_ext = jnp.concatenate([times_dif, times_dif[-1:]], axis=0)
    s0, s1, s2 = sp_ext[2:], sp_ext[1:-1], sp_ext[:-2]
    d0, d1, d2 = dif_ext[2:], dif_ext[1:-1], dif_ext[:-2]
    dir_p = jnp.transpose(dir, (2, 0, 1))
    bias_n = jnp.concatenate([bias[:, 1:], bias[:, -1:]], axis=1)
    ts = time_shift.reshape(1, 3)
    types2 = types.reshape(_N, 1)

    grid = (tp + _R - 1) // _R
    row_spec3 = pl.BlockSpec((_R, 3), lambda i: (i, 0))
    row_spec1 = pl.BlockSpec((_R, 1), lambda i: (i, 0))
    out = pl.pallas_call(
        functools.partial(_block_kernel, tp=tp),
        grid=(grid,),
        in_specs=[
            pl.BlockSpec((1, 3), lambda i: (0, 0)),
            pl.BlockSpec((1, 3), lambda i: (0, 0)),
            pl.BlockSpec((_N, 1), lambda i: (0, 0)),
            row_spec3, row_spec3, row_spec3,
            row_spec1, row_spec1, row_spec1,
            pl.BlockSpec((3, _R, _N), lambda i: (0, i, 0)),
            pl.BlockSpec((_R, _N), lambda i: (i, 0)),
            pl.BlockSpec((_R, _N), lambda i: (i, 0)),
            pl.BlockSpec((6, _R), lambda i: (0, i)),
            pl.BlockSpec((6, _R), lambda i: (0, i)),
        ],
        out_specs=pl.BlockSpec((1, _R), lambda i: (0, i)),
        out_shape=jax.ShapeDtypeStruct((1, tp), jnp.float32),
    )(ts, bias_shift, types2, s0, s1, s2, d0, d1, d2, dir_p, mes, weight,
      bias, bias_n)
    loss = out.reshape(tp)
    return jnp.concatenate([jnp.zeros((1,), jnp.float32), loss], axis=0)


# speedup vs baseline: 7.1871x; 1.1738x over previous
import functools

import numpy as np
import jax
import jax.numpy as jnp
from jax.experimental import pallas as pl

_R = 1024
_N = 256


def _block_kernel(ts_ref, bsh_ref, types_ref,
                  s0_ref, s1_ref, s2_ref, d0_ref, d1_ref, d2_ref,
                  dir_ref, mes_ref, w_ref, bc_ref, bn_ref,
                  out_ref, *, tp):
    ts0 = ts_ref[0, 0]
    ts1 = ts_ref[0, 1]
    ts2 = ts_ref[0, 2]
    inv_sum = 1.0 / (ts0 + ts1 + ts2)
    sm_t = (s0_ref[...] * (1000.0 / d0_ref[...]) * ts0
            + s1_ref[...] * (1000.0 / d1_ref[...]) * ts1
            + s2_ref[...] * (1000.0 / d2_ref[...]) * ts2) * inv_sum
    sm_t = sm_t + bsh_ref[...] * 0.01
    row0 = pl.program_id(0) * _R
    lane_ok = (jax.lax.broadcasted_iota(jnp.int32, (3, _R), 1) + row0 < tp)
    sm_t = jnp.where(lane_ok, sm_t, 0.0)

    ident = (jax.lax.broadcasted_iota(jnp.int32, (_R, _R), 0)
             == jax.lax.broadcasted_iota(jnp.int32, (_R, _R), 1)
             ).astype(jnp.float32)
    tdims = (((0,), (0,)), ((), ()))
    sm = jax.lax.dot_general(ident, sm_t, (((1,), (1,)), ((), ())),
                             preferred_element_type=jnp.float32)

    c = (dir_ref[0] * sm[:, 0:1] + dir_ref[1] * sm[:, 1:2]
         + dir_ref[2] * sm[:, 2:3]) - mes_ref[...]

    valid = (jax.lax.broadcasted_iota(jnp.int32, (_R, _N), 0) + row0
             < tp)
    c = jnp.where(valid, c, 0.0)
    w_in = jnp.where(valid, w_ref[...], 0.0)
    c_t = jax.lax.dot_general(c, ident, tdims,
                              preferred_element_type=jnp.float32)
    w_t = jax.lax.dot_general(w_in, ident, tdims,
                              preferred_element_type=jnp.float32)

    kcol = jax.lax.broadcasted_iota(jnp.int32, (_N, 6), 1)
    onehot = (types_ref[...] == kcol).astype(jnp.float32)
    b_t = jnp.dot(onehot, bc_ref[...], preferred_element_type=jnp.float32)

    mes_est = c_t + b_t

    x = mes_est + (w_t == 0.0).astype(jnp.float32) * 10000000.0
    u = jax.lax.bitcast_convert_type(x, jnp.int32)
    skey = u ^ ((u >> 31) & jnp.int32(0x7FFFFFFF))
    ind = jnp.sum((w_t > 0.0).astype(jnp.int32), axis=0, keepdims=True)
    kf = (ind // 2).astype(jnp.float32)
    p = jnp.full((1, _R), np.int32(-2147483648), jnp.int32)
    for bit in range(31, -1, -1):
        inc = np.int32(-2147483648) if bit == 31 else np.int32(1 << bit)
        cand = p + inc
        cnt = jnp.sum(jnp.where(skey < cand, 1.0, 0.0), axis=0,
                      keepdims=True)
        p = jnp.where(cnt <= kf, cand, p)
    med_u = p ^ ((p >> 31) & jnp.int32(0x7FFFFFFF))
    med = jax.lax.bitcast_convert_type(med_u, jnp.float32)
    med = med * (ind > 0).astype(jnp.float32)

    row = jnp.sum(jnp.abs(mes_est - med) * w_t, axis=0, keepdims=True)
    row = row * np.float32(1.0 / _N)
    bias_loss = jnp.sum(jnp.abs(bn_ref[...] - bc_ref[...]), axis=0,
                        keepdims=True)
    out_ref[...] = row + bias_loss


def kernel(speed, quats, times_dif, dir, mes, weight, bias, bias_shift,
           time_shift, types):
    del quats
    tp = dir.shape[0]
    sp_ext = jnp.concatenate([jnp.transpose(speed),
                              jnp.transpose(speed[-1:])], axis=1)
    dif_ext = jnp.concatenate([jnp.transpose(times_dif),
                               jnp.transpose(times_dif[-1:])], axis=1)
    s0, s1, s2 = sp_ext[:, 2:], sp_ext[:, 1:-1], sp_ext[:, :-2]
    d0, d1, d2 = dif_ext[:, 2:], dif_ext[:, 1:-1], dif_ext[:, :-2]
    dir_p = jnp.transpose(dir, (2, 0, 1))
    bias_n = jnp.concatenate([bias[:, 1:], bias[:, -1:]], axis=1)
    ts = time_shift.reshape(1, 3)
    bsh = bias_shift.reshape(3, 1)
    types2 = types.reshape(_N, 1)

    grid = (tp + _R - 1) // _R
    row_spec3 = pl.BlockSpec((3, _R), lambda i: (0, i))
    row_spec1 = pl.BlockSpec((1, _R), lambda i: (0, i))
    out = pl.pallas_call(
        functools.partial(_block_kernel, tp=tp),
        grid=(grid,),
        in_specs=[
            pl.BlockSpec((1, 3), lambda i: (0, 0)),
            pl.BlockSpec((3, 1), lambda i: (0, 0)),
            pl.BlockSpec((_N, 1), lambda i: (0, 0)),
            row_spec3, row_spec3, row_spec3,
            row_spec1, row_spec1, row_spec1,
            pl.BlockSpec((3, _R, _N), lambda i: (0, i, 0)),
            pl.BlockSpec((_R, _N), lambda i: (i, 0)),
            pl.BlockSpec((_R, _N), lambda i: (i, 0)),
            pl.BlockSpec((6, _R), lambda i: (0, i)),
            pl.BlockSpec((6, _R), lambda i: (0, i)),
        ],
        out_specs=pl.BlockSpec((1, _R), lambda i: (0, i)),
        out_shape=jax.ShapeDtypeStruct((1, tp), jnp.float32),
    )(ts, bsh, types2, s0, s1, s2, d0, d1, d2, dir_p, mes, weight,
      bias, bias_n)
    loss = out.reshape(tp)
    return jnp.concatenate([jnp.zeros((1,), jnp.float32), loss], axis=0)


# speedup vs baseline: 9.0772x; 1.2630x over previous
import functools

import numpy as np
import jax
import jax.numpy as jnp
from jax.experimental import pallas as pl

_R = 1024
_N = 256


def _block_kernel(ts_ref, bsh_ref, types_ref,
                  s0_ref, s1_ref, s2_ref, d0_ref, d1_ref, d2_ref,
                  dir_ref, mes_ref, w_ref, bc_ref, bn_ref,
                  out_ref, *, tp):
    ts0 = ts_ref[0, 0]
    ts1 = ts_ref[0, 1]
    ts2 = ts_ref[0, 2]
    inv_sum = 1.0 / (ts0 + ts1 + ts2)
    sm_t = (s0_ref[...] * (1000.0 / d0_ref[...]) * ts0
            + s1_ref[...] * (1000.0 / d1_ref[...]) * ts1
            + s2_ref[...] * (1000.0 / d2_ref[...]) * ts2) * inv_sum
    sm_t = sm_t + bsh_ref[...] * 0.01

    sm = jnp.transpose(sm_t)

    c = (dir_ref[0] * sm[:, 0:1] + dir_ref[1] * sm[:, 1:2]
         + dir_ref[2] * sm[:, 2:3]) - mes_ref[...]

    c_t = jnp.transpose(c)
    w_t = jnp.transpose(w_ref[...])

    kcol = jax.lax.broadcasted_iota(jnp.int32, (_N, 6), 1)
    onehot = (types_ref[...] == kcol).astype(jnp.float32)
    b_t = jnp.dot(onehot, bc_ref[...], preferred_element_type=jnp.float32)

    mes_est = c_t + b_t

    x = mes_est + (w_t == 0.0).astype(jnp.float32) * 10000000.0
    u = jax.lax.bitcast_convert_type(x, jnp.int32)
    skey = u ^ ((u >> 31) & jnp.int32(0x7FFFFFFF))
    ind = jnp.sum((w_t > 0.0).astype(jnp.int32), axis=0, keepdims=True)
    kf = (ind // 2).astype(jnp.float32)
    p = jnp.full((1, _R), np.int32(-2147483648), jnp.int32)
    for bit in range(31, -1, -1):
        inc = np.int32(-2147483648) if bit == 31 else np.int32(1 << bit)
        cand = p + inc
        cnt = jnp.sum(jnp.where(skey < cand, 1.0, 0.0), axis=0,
                      keepdims=True)
        p = jnp.where(cnt <= kf, cand, p)
    med_u = p ^ ((p >> 31) & jnp.int32(0x7FFFFFFF))
    med = jax.lax.bitcast_convert_type(med_u, jnp.float32)
    med = med * (ind > 0).astype(jnp.float32)

    row = jnp.sum(jnp.abs(mes_est - med) * w_t, axis=0, keepdims=True)
    row = row * np.float32(1.0 / _N)
    bias_loss = jnp.sum(jnp.abs(bn_ref[...] - bc_ref[...]), axis=0,
                        keepdims=True)
    out_ref[...] = row + bias_loss


def kernel(speed, quats, times_dif, dir, mes, weight, bias, bias_shift,
           time_shift, types):
    del quats
    tp = dir.shape[0]
    sp_ext = jnp.concatenate([jnp.transpose(speed),
                              jnp.transpose(speed[-1:])], axis=1)
    dif_ext = jnp.concatenate([jnp.transpose(times_dif),
                               jnp.transpose(times_dif[-1:])], axis=1)
    s0, s1, s2 = sp_ext[:, 2:], sp_ext[:, 1:-1], sp_ext[:, :-2]
    d0, d1, d2 = dif_ext[:, 2:], dif_ext[:, 1:-1], dif_ext[:, :-2]
    dir_p = jnp.transpose(dir, (2, 0, 1))
    bias_n = jnp.concatenate([bias[:, 1:], bias[:, -1:]], axis=1)
    ts = time_shift.reshape(1, 3)
    bsh = bias_shift.reshape(3, 1)
    types2 = types.reshape(_N, 1)

    grid = (tp + _R - 1) // _R
    row_spec3 = pl.BlockSpec((3, _R), lambda i: (0, i))
    row_spec1 = pl.BlockSpec((1, _R), lambda i: (0, i))
    out = pl.pallas_call(
        functools.partial(_block_kernel, tp=tp),
        grid=(grid,),
        in_specs=[
            pl.BlockSpec((1, 3), lambda i: (0, 0)),
            pl.BlockSpec((3, 1), lambda i: (0, 0)),
            pl.BlockSpec((_N, 1), lambda i: (0, 0)),
            row_spec3, row_spec3, row_spec3,
            row_spec1, row_spec1, row_spec1,
            pl.BlockSpec((3, _R, _N), lambda i: (0, i, 0)),
            pl.BlockSpec((_R, _N), lambda i: (i, 0)),
            pl.BlockSpec((_R, _N), lambda i: (i, 0)),
            pl.BlockSpec((6, _R), lambda i: (0, i)),
            pl.BlockSpec((6, _R), lambda i: (0, i)),
        ],
        out_specs=pl.BlockSpec((1, _R), lambda i: (0, i)),
        out_shape=jax.ShapeDtypeStruct((1, tp), jnp.float32),
    )(ts, bsh, types2, s0, s1, s2, d0, d1, d2, dir_p, mes, weight,
      bias, bias_n)
    loss = out.reshape(tp)
    return jnp.concatenate([jnp.zeros((1,), jnp.float32), loss], axis=0)


# speedup vs baseline: 10.4613x; 1.1525x over previous
import functools

import numpy as np
import jax
import jax.numpy as jnp
from jax.experimental import pallas as pl

_R = 2048
_N = 256


def _block_kernel(ts_ref, bsh_ref, types_ref,
                  s0_ref, s1_ref, s2_ref, d0_ref, d1_ref, d2_ref,
                  dir_ref, mes_ref, w_ref, bc_ref, bn_ref,
                  out_ref, *, tp):
    ts0 = ts_ref[0, 0]
    ts1 = ts_ref[0, 1]
    ts2 = ts_ref[0, 2]
    inv_sum = 1.0 / (ts0 + ts1 + ts2)
    sm_t = (s0_ref[...] * (1000.0 / d0_ref[...]) * ts0
            + s1_ref[...] * (1000.0 / d1_ref[...]) * ts1
            + s2_ref[...] * (1000.0 / d2_ref[...]) * ts2) * inv_sum
    sm_t = sm_t + bsh_ref[...] * 0.01

    sm = jnp.transpose(sm_t)

    c = (dir_ref[0] * sm[:, 0:1] + dir_ref[1] * sm[:, 1:2]
         + dir_ref[2] * sm[:, 2:3]) - mes_ref[...]

    c_t = jnp.transpose(c)
    w_t = jnp.transpose(w_ref[...])

    kcol = jax.lax.broadcasted_iota(jnp.int32, (_N, 6), 1)
    onehot = (types_ref[...] == kcol).astype(jnp.float32)
    b_t = jnp.dot(onehot, bc_ref[...], preferred_element_type=jnp.float32)

    mes_est = c_t + b_t

    x = mes_est + (w_t == 0.0).astype(jnp.float32) * 10000000.0
    u = jax.lax.bitcast_convert_type(x, jnp.int32)
    skey = u ^ ((u >> 31) & jnp.int32(0x7FFFFFFF))
    ind = jnp.sum((w_t > 0.0).astype(jnp.int32), axis=0, keepdims=True)
    kf = (ind // 2).astype(jnp.float32)
    p = jnp.full((1, _R), np.int32(-2147483648), jnp.int32)
    for bit in range(31, 7, -1):
        inc = np.int32(-2147483648) if bit == 31 else np.int32(1 << bit)
        cand = p + inc
        cnt = jnp.sum(jnp.where(skey < cand, 1.0, 0.0), axis=0,
                      keepdims=True)
        p = jnp.where(cnt <= kf, cand, p)
    diff = skey - p
    in_grp = (diff >= 0) & (diff < 256)
    med_p = jnp.min(jnp.where(in_grp, skey, np.int32(2147483647)), axis=0,
                    keepdims=True)
    med_u = med_p ^ ((med_p >> 31) & jnp.int32(0x7FFFFFFF))
    med = jax.lax.bitcast_convert_type(med_u, jnp.float32)
    med = med * (ind > 0).astype(jnp.float32)

    row = jnp.sum(jnp.abs(mes_est - med) * w_t, axis=0, keepdims=True)
    row = row * np.float32(1.0 / _N)
    bias_loss = jnp.sum(jnp.abs(bn_ref[...] - bc_ref[...]), axis=0,
                        keepdims=True)
    out_ref[...] = row + bias_loss


def kernel(speed, quats, times_dif, dir, mes, weight, bias, bias_shift,
           time_shift, types):
    del quats
    tp = dir.shape[0]
    sp_ext = jnp.concatenate([jnp.transpose(speed),
                              jnp.transpose(speed[-1:])], axis=1)
    dif_ext = jnp.concatenate([jnp.transpose(times_dif),
                               jnp.transpose(times_dif[-1:])], axis=1)
    s0, s1, s2 = sp_ext[:, 2:], sp_ext[:, 1:-1], sp_ext[:, :-2]
    d0, d1, d2 = dif_ext[:, 2:], dif_ext[:, 1:-1], dif_ext[:, :-2]
    dir_p = jnp.transpose(dir, (2, 0, 1))
    bias_n = jnp.concatenate([bias[:, 1:], bias[:, -1:]], axis=1)
    ts = time_shift.reshape(1, 3)
    bsh = bias_shift.reshape(3, 1)
    types2 = types.reshape(_N, 1)

    grid = (tp + _R - 1) // _R
    row_spec3 = pl.BlockSpec((3, _R), lambda i: (0, i))
    row_spec1 = pl.BlockSpec((1, _R), lambda i: (0, i))
    out = pl.pallas_call(
        functools.partial(_block_kernel, tp=tp),
        grid=(grid,),
        in_specs=[
            pl.BlockSpec((1, 3), lambda i: (0, 0)),
            pl.BlockSpec((3, 1), lambda i: (0, 0)),
            pl.BlockSpec((_N, 1), lambda i: (0, 0)),
            row_spec3, row_spec3, row_spec3,
            row_spec1, row_spec1, row_spec1,
            pl.BlockSpec((3, _R, _N), lambda i: (0, i, 0)),
            pl.BlockSpec((_R, _N), lambda i: (i, 0)),
            pl.BlockSpec((_R, _N), lambda i: (i, 0)),
            pl.BlockSpec((6, _R), lambda i: (0, i)),
            pl.BlockSpec((6, _R), lambda i: (0, i)),
        ],
        out_specs=pl.BlockSpec((1, _R), lambda i: (0, i)),
        out_shape=jax.ShapeDtypeStruct((1, tp), jnp.float32),
    )(ts, bsh, types2, s0, s1, s2, d0, d1, d2, dir_p, mes, weight,
      bias, bias_n)
    loss = out.reshape(tp)
    return jnp.concatenate([jnp.zeros((1,), jnp.float32), loss], axis=0)
